# Initial kernel scaffold; baseline (speedup 1.0000x reference)
import jax
import jax.numpy as jnp
from jax.experimental import pallas as pl


def kernel(fv_features, points_img, proj_masks, points_img_far, proj_masks_far):
    raise NotImplementedError("write your pallas kernel here")


# speedup vs baseline: 3.0983x; 3.0983x over previous
import functools

import jax
import jax.numpy as jnp
from jax.experimental import pallas as pl

_VOXEL = (0.2, 0.2)
_PC_MIN = (-25.6, -25.6)
_NX = 256
_NY = 256

_TQ = 256
_TK = 2048

_BIG_I = 2**30
_DIAG_XLA = False


def _three_nn_body(q_ref, k_ref, dist_ref, idx_ref, *, n_known, tk):
    q = q_ref[0]
    k = k_ref[0]
    tq = q.shape[0]
    q3 = q[:, 0:3].astype(jnp.bfloat16)
    qn = q[:, 3:4]

    inf = jnp.float32(jnp.inf)
    v1 = jnp.full((tq, 1), inf, jnp.float32)
    v2 = jnp.full((tq, 1), inf, jnp.float32)
    v3 = jnp.full((tq, 1), inf, jnp.float32)
    i1 = jnp.zeros((tq, 1), jnp.int32)
    i2 = jnp.zeros((tq, 1), jnp.int32)
    i3 = jnp.zeros((tq, 1), jnp.int32)

    n_chunks = n_known // tk
    for c in range(n_chunks):
        kc = k[0:3, c * tk:(c + 1) * tk].astype(jnp.bfloat16)
        knc = k[3:4, c * tk:(c + 1) * tk]
        mm = jax.lax.dot_general(q3, kc, (((1,), (0,)), ((), ())),
                                 preferred_element_type=jnp.float32)
        d = (qn + knc) - 2.0 * mm
        iota = jax.lax.broadcasted_iota(jnp.int32, (tq, tk), 1) + c * tk
        for p in range(3):
            m = jnp.min(d, axis=1, keepdims=True)
            im = jnp.min(jnp.where(d == m, iota, _BIG_I), axis=1, keepdims=True)
            if p < 2:
                d = jnp.where(iota == im, inf, d)
            b1 = m < v1
            b2 = m < v2
            b3 = m < v3
            v1n = jnp.where(b1, m, v1)
            i1n = jnp.where(b1, im, i1)
            v2n = jnp.where(b1, v1, jnp.where(b2, m, v2))
            i2n = jnp.where(b1, i1, jnp.where(b2, im, i2))
            v3n = jnp.where(b2, v2, jnp.where(b3, m, v3))
            i3n = jnp.where(b2, i2, jnp.where(b3, im, i3))
            v1, v2, v3, i1, i2, i3 = v1n, v2n, v3n, i1n, i2n, i3n

    dist_ref[0] = jnp.concatenate([v1, v2, v3], axis=1)
    idx_ref[0] = jnp.concatenate([i1, i2, i3], axis=1)


def _three_nn_pallas(q_aug, k_aug, *, interpret=False):
    b, n, _ = q_aug.shape
    m = k_aug.shape[2]
    tq = min(_TQ, n)
    tk = min(_TK, m)
    grid = (b, n // tq)
    return pl.pallas_call(
        functools.partial(_three_nn_body, n_known=m, tk=tk),
        grid=grid,
        in_specs=[
            pl.BlockSpec((1, tq, 4), lambda bi, qi: (bi, qi, 0)),
            pl.BlockSpec((1, 4, m), lambda bi, qi: (bi, 0, 0)),
        ],
        out_specs=[
            pl.BlockSpec((1, tq, 3), lambda bi, qi: (bi, qi, 0)),
            pl.BlockSpec((1, tq, 3), lambda bi, qi: (bi, qi, 0)),
        ],
        out_shape=[
            jax.ShapeDtypeStruct((b, n, 3), jnp.float32),
            jax.ShapeDtypeStruct((b, n, 3), jnp.int32),
        ],
        interpret=interpret,
    )(q_aug, k_aug)


def _cells(points):
    xi = jnp.clip(jnp.floor((points[..., 0] - _PC_MIN[0]) / _VOXEL[0]).astype(jnp.int32), 0, _NX - 1)
    yi = jnp.clip(jnp.floor((points[..., 1] - _PC_MIN[1]) / _VOXEL[1]).astype(jnp.int32), 0, _NY - 1)
    return yi * _NX + xi


def kernel(fv_features, points_img, proj_masks, points_img_far, proj_masks_far):
    b, c, h, w = fv_features.shape
    n = h * w
    feats = jnp.transpose(fv_features, (0, 2, 3, 1)).reshape(b, n, c)
    pts = jnp.transpose(points_img[:, :3], (0, 2, 3, 1)).reshape(b, n, 3)
    pts_far = jnp.transpose(points_img_far[:, :3], (0, 2, 3, 1)).reshape(b, n, 3)

    qn = (pts_far ** 2).sum(-1)
    kn = (pts ** 2).sum(-1)
    q_aug = jnp.concatenate([pts_far, qn[..., None]], axis=-1)
    k_aug = jnp.concatenate(
        [jnp.transpose(pts, (0, 2, 1)), kn[:, None, :]], axis=1)

    if _DIAG_XLA:
        def _nn(qb, kb, qnb, knb):
            d = qnb[:, None] + knb[None, :] - 2.0 * (qb @ kb.T)
            negd, ix = jax.lax.top_k(-d, 3)
            return -negd, ix
        dist, idx = jax.vmap(_nn)(pts_far, pts, qn, kn)
    else:
        dist, idx = _three_nn_pallas(q_aug, k_aug)

    dist = jnp.maximum(dist, 0.0)
    recip = 1.0 / (dist + 1e-8)
    weight = recip / recip.sum(axis=-1, keepdims=True)

    gathered = jax.vmap(lambda f, i: f[i])(feats, idx)
    interp = (gathered * weight[..., None]).sum(axis=2)

    cells_known = _cells(pts)
    cells_far = _cells(pts_far)

    m_w = proj_masks.reshape(b, n)
    mf_w = proj_masks_far.reshape(b, n)

    def project(cells_k, cells_f, f_k, f_f, w_k, w_f):
        lin = jnp.concatenate([cells_k, cells_f], axis=0)
        fall = jnp.concatenate([f_k, f_f], axis=0)
        wall = jnp.concatenate([w_k, w_f], axis=0)
        sums = jnp.zeros((_NY * _NX, c), jnp.float32).at[lin].add(fall * wall[:, None])
        cnt = jnp.zeros((_NY * _NX,), jnp.float32).at[lin].add(wall)
        bev = sums / jnp.maximum(cnt, 1.0)[:, None]
        return jnp.transpose(bev.reshape(_NY, _NX, c), (2, 0, 1))

    return jax.vmap(project)(cells_known, cells_far, feats, interp, m_w, mf_w)
